# Initial kernel scaffold; baseline (speedup 1.0000x reference)
import jax
import jax.numpy as jnp
from jax.experimental import pallas as pl


def kernel(gh_label, gah_label, p_gh, p_gah, mask):
    raise NotImplementedError("write your pallas kernel here")


# speedup vs baseline: 25.8199x; 25.8199x over previous
import functools

import jax
import jax.numpy as jnp
from jax import lax
from jax.experimental import pallas as pl
from jax.experimental.pallas import tpu as pltpu

B, H, W = 8, 384, 384
HW = H * W
CW = 6144
NCH = HW // CW
N_F = float(HW)
ONE_BITS = 0x3F800000


def _body(ghl, gal, pgh, pga, msk, out, pl_s, np_a, sp_a, sn_a):
    i = pl.program_id(0)

    @pl.when(i == 0)
    def _init():
        np_a[...] = jnp.zeros_like(np_a)
        sp_a[...] = jnp.zeros_like(sp_a)
        sn_a[...] = jnp.zeros_like(sn_a)

    m = msk[...]
    for row0, lab_ref, prd_ref in ((0, ghl, pgh), (8, gal, pga)):
        lab = lab_ref[...]
        diff = prd_ref[...] - lab
        plv = diff * diff * m
        pos = lab >= 0.1
        posf = pos.astype(jnp.float32)
        np_a[row0:row0 + 8, 0:1] = np_a[row0:row0 + 8, 0:1] + jnp.sum(
            posf, axis=1, keepdims=True)
        sp_a[row0:row0 + 8, 0:1] = sp_a[row0:row0 + 8, 0:1] + jnp.sum(
            plv * posf, axis=1, keepdims=True)
        sn_a[row0:row0 + 8, 0:1] = sn_a[row0:row0 + 8, 0:1] + jnp.sum(
            plv * (1.0 - posf), axis=1, keepdims=True)
        pl_s[i, row0:row0 + 8, :] = jnp.where(pos, -1.0, plv)

    @pl.when(i == NCH - 1)
    def _finish():
        n_pos = np_a[:, 0:1]
        sum_pos = sp_a[:, 0:1]
        sum_neg = sn_a[:, 0:1]
        n_neg = N_F - n_pos
        kf = jnp.where(n_pos > 0.0, 3.0 * n_pos, 500.0)

        def count_ge(mid):
            def inner(j, c):
                vi = lax.bitcast_convert_type(pl_s[j], jnp.int32)
                return c + jnp.sum((vi >= mid).astype(jnp.float32),
                                   axis=1, keepdims=True)
            return lax.fori_loop(0, NCH, inner,
                                 jnp.zeros((16, 1), jnp.float32))

        def search_step(_, carry):
            lo, hi = carry
            mid = lo + ((hi - lo + 1) >> 1)
            take = count_ge(mid) >= kf
            return jnp.where(take, mid, lo), jnp.where(take, hi, mid - 1)

        lo0 = jnp.zeros((16, 1), jnp.int32)
        hi0 = jnp.full((16, 1), ONE_BITS, jnp.int32)
        t_int, _ = lax.fori_loop(0, 30, search_step, (lo0, hi0))

        def tail(j, c):
            sgt, cgt = c
            v = pl_s[j]
            vi = lax.bitcast_convert_type(v, jnp.int32)
            gt = vi > t_int
            sgt = sgt + jnp.sum(jnp.where(gt, v, 0.0), axis=1, keepdims=True)
            cgt = cgt + jnp.sum(gt.astype(jnp.float32), axis=1, keepdims=True)
            return sgt, cgt

        z = jnp.zeros((16, 1), jnp.float32)
        sum_gt, cnt_gt = lax.fori_loop(0, NCH, tail, (z, z))
        t_val = lax.bitcast_convert_type(t_int, jnp.float32)
        topk_sum = sum_gt + (kf - cnt_gt) * t_val
        topk_mean = topk_sum / jnp.maximum(kf, 1.0)

        posi = sum_pos / jnp.maximum(n_pos, 1.0)
        negall = sum_neg / jnp.maximum(n_neg, 1.0)
        nega = jnp.where(n_neg < 3.0 * n_pos, negall, topk_mean)
        per = jnp.where(n_pos > 0.0, posi + nega, topk_mean)
        out[...] = jnp.sum(per, keepdims=True) / float(B)


@jax.jit
def kernel(gh_label, gah_label, p_gh, p_gah, mask):
    ghl = gh_label.reshape(B, HW)
    gal = gah_label.reshape(B, HW)
    pgh = p_gh.reshape(B, HW)
    pga = p_gah.reshape(B, HW)
    msk = mask.reshape(B, HW)

    spec = pl.BlockSpec((B, CW), lambda i: (0, i))
    out = pl.pallas_call(
        _body,
        grid=(NCH,),
        in_specs=[spec, spec, spec, spec, spec],
        out_specs=pl.BlockSpec((1, 1), lambda i: (0, 0)),
        out_shape=jax.ShapeDtypeStruct((1, 1), jnp.float32),
        scratch_shapes=[
            pltpu.VMEM((NCH, 2 * B, CW), jnp.float32),
            pltpu.VMEM((2 * B, 128), jnp.float32),
            pltpu.VMEM((2 * B, 128), jnp.float32),
            pltpu.VMEM((2 * B, 128), jnp.float32),
        ],
    )(ghl, gal, pgh, pga, msk)
    return out[0, 0]
